# baseline (speedup 1.0000x reference)
import functools

import jax
import jax.numpy as jnp
from jax import lax
from jax.experimental import pallas as pl
from jax.experimental.pallas import tpu as pltpu
from jax.experimental.pallas import tpu_sc as plsc

NUM_CORES = 2
NUM_SUBCORES = 16
NUM_WORKERS = NUM_CORES * NUM_SUBCORES


def kernel(h, r, t, entity_table, relation_table):
    batch = h.shape[0]
    dim = entity_table.shape[1]
    assert batch % (8 * NUM_WORKERS) == 0
    b_per_w = batch // NUM_WORKERS

    mesh = plsc.VectorSubcoreMesh(core_axis_name="c", subcore_axis_name="s")
    out_sds = jax.ShapeDtypeStruct((batch, dim), jnp.float32)

    @functools.partial(
        pl.kernel,
        out_type=(out_sds, out_sds, out_sds),
        mesh=mesh,
        compiler_params=pltpu.CompilerParams(use_tc_tiling_on_sc=False),
        scratch_types=[
            pltpu.VMEM((b_per_w,), jnp.int32),
            pltpu.VMEM((b_per_w,), jnp.int32),
            pltpu.VMEM((b_per_w,), jnp.int32),
            pltpu.VMEM((b_per_w, dim), jnp.float32),
            pltpu.VMEM((b_per_w, dim), jnp.float32),
            pltpu.VMEM((b_per_w, dim), jnp.float32),
            pltpu.SemaphoreType.DMA,
            pltpu.SemaphoreType.DMA,
            pltpu.SemaphoreType.DMA,
        ],
    )
    def emb_kernel(h_hbm, r_hbm, t_hbm, etab, rtab, h_out, r_out, t_out,
                   hi_v, ri_v, ti_v, hrows, rrows, trows, sem_h, sem_r, sem_t):
        wid = lax.axis_index("s") * NUM_CORES + lax.axis_index("c")
        base = wid * b_per_w
        pltpu.sync_copy(h_hbm.at[pl.ds(base, b_per_w)], hi_v)
        copy_h = pltpu.async_copy(etab.at[hi_v], hrows, sem_h)
        pltpu.sync_copy(r_hbm.at[pl.ds(base, b_per_w)], ri_v)
        copy_r = pltpu.async_copy(rtab.at[ri_v], rrows, sem_r)
        pltpu.sync_copy(t_hbm.at[pl.ds(base, b_per_w)], ti_v)
        copy_t = pltpu.async_copy(etab.at[ti_v], trows, sem_t)
        copy_h.wait()
        pltpu.sync_copy(hrows, h_out.at[pl.ds(base, b_per_w)])
        copy_r.wait()
        pltpu.sync_copy(rrows, r_out.at[pl.ds(base, b_per_w)])
        copy_t.wait()
        pltpu.sync_copy(trows, t_out.at[pl.ds(base, b_per_w)])

    return emb_kernel(h, r, t, entity_table, relation_table)


# speedup vs baseline: 1.4877x; 1.4877x over previous
import functools

import jax
import jax.numpy as jnp
from jax import lax
from jax.experimental import pallas as pl
from jax.experimental.pallas import tpu as pltpu
from jax.experimental.pallas import tpu_sc as plsc

NUM_CORES = 2
NUM_SUBCORES = 16
NUM_WORKERS = NUM_CORES * NUM_SUBCORES
CHUNK = 32


def kernel(h, r, t, entity_table, relation_table):
    batch = h.shape[0]
    dim = entity_table.shape[1]
    assert batch % (8 * NUM_WORKERS) == 0
    b_per_w = batch // NUM_WORKERS
    n_chunks = b_per_w // CHUNK

    mesh = plsc.VectorSubcoreMesh(core_axis_name="c", subcore_axis_name="s")
    out_sds = jax.ShapeDtypeStruct((batch, dim), jnp.float32)

    @functools.partial(
        pl.kernel,
        out_type=(out_sds, out_sds, out_sds),
        mesh=mesh,
        scratch_types=[
            pltpu.VMEM((b_per_w,), jnp.int32),
            pltpu.VMEM((CHUNK, dim), jnp.float32),
            pltpu.SemaphoreType.DMA,
        ],
    )
    def emb_kernel(h_hbm, r_hbm, t_hbm, etab, rtab, h_out, r_out, t_out,
                   idx_v, rows_v, sem):
        wid = lax.axis_index("s") * NUM_CORES + lax.axis_index("c")
        base = wid * b_per_w

        def run_lookup(src_hbm, table, out_hbm):
            pltpu.sync_copy(src_hbm.at[pl.ds(base, b_per_w)], idx_v)

            def chunk_body(c, _):
                cb = c * CHUNK
                copies = []
                for g in range(CHUNK // 16):
                    ivec = idx_v[pl.ds(cb + g * 16, 16)]
                    for k in range(16):
                        copies.append(
                            pltpu.async_copy(table.at[ivec[k]],
                                             rows_v.at[g * 16 + k], sem))
                for cp in copies:
                    cp.wait()
                pltpu.sync_copy(rows_v, out_hbm.at[pl.ds(base + cb, CHUNK)])
                return 0
            lax.fori_loop(0, n_chunks, chunk_body, 0)

        run_lookup(h_hbm, etab, h_out)
        run_lookup(r_hbm, rtab, r_out)
        run_lookup(t_hbm, etab, t_out)

    return emb_kernel(h, r, t, entity_table, relation_table)
